# Initial kernel scaffold; baseline (speedup 1.0000x reference)
import jax
import jax.numpy as jnp
from jax.experimental import pallas as pl


def kernel(x, edge_index, edge_attr, node_w, node_b, edge_w, edge_b, mp0_msg_w1, mp0_msg_b1, mp0_msg_w2, mp0_msg_b2, mp0_upd_w1, mp0_upd_b1, mp0_upd_w2, mp0_upd_b2, mp1_msg_w1, mp1_msg_b1, mp1_msg_w2, mp1_msg_b2, mp1_upd_w1, mp1_upd_b1, mp1_upd_w2, mp1_upd_b2, final_w, final_b):
    raise NotImplementedError("write your pallas kernel here")


# speedup vs baseline: 3.5487x; 3.5487x over previous
import functools

import jax
import jax.numpy as jnp
from jax import lax
from jax.experimental import pallas as pl
from jax.experimental.pallas import tpu as pltpu
from jax.experimental.pallas import tpu_sc as plsc

N = 10000
NP = 10240
E = 320000
DIN = 128
DE = 16
F = 128

NC = 2
NS = 16
NW = NC * NS
EPW = E // NW
CHUNK = 80
NCHUNKS = EPW // CHUNK
STRIPE = NP // NS

BN = 512
GRID_N = NP // BN
BE = 4000
GRID_E = E // BE


def _relu(t):
    return jnp.maximum(t, 0.0)


def _dot(a, b):
    return jnp.dot(a, b, preferred_element_type=jnp.float32)


def _node_prep_body(x_ref, nw_ref, nb_ref, mw1_ref, mb1_ref, x1_ref, hx0_ref):
    x1 = _dot(x_ref[...], nw_ref[...]) + nb_ref[...]
    x1_ref[...] = x1
    hx0_ref[...] = _relu(_dot(x1, mw1_ref[...]) + mb1_ref[...])


def _edge_prep_body(ea_ref, ew_ref, eb_ref, mw10_ref, mb10_ref,
                    mw11_ref, mb11_ref, he0_ref, he1_ref):
    ea = ea_ref[...]
    ew = ew_ref[...]
    eb = eb_ref[...]
    a0 = _dot(ew, mw10_ref[...])
    c0 = _dot(eb, mw10_ref[...]) + mb10_ref[...]
    he0_ref[...] = _relu(_dot(ea, a0) + c0)
    a1 = _dot(ew, mw11_ref[...])
    c1 = _dot(eb, mw11_ref[...]) + mb11_ref[...]
    he1_ref[...] = _relu(_dot(ea, a1) + c1)


def _mid_body(x1_ref, ga_ref, gb_ref, cnt_ref,
              mw2_ref, mb2_ref, uw1_ref, ub1_ref, uw2_ref, ub2_ref,
              mw1n_ref, mb1n_ref,
              x2_ref, hx1_ref, m_ref):
    cnt = cnt_ref[...]
    inv = 1.0 / jnp.maximum(cnt, 1.0)
    ind = (cnt > 0.0).astype(jnp.float32)
    g = (ga_ref[...] + gb_ref[...]) * inv
    aggr = _dot(g, mw2_ref[...]) + 2.0 * ind * mb2_ref[...]
    x1 = x1_ref[...]
    uw1 = uw1_ref[...]
    ub1 = ub1_ref[...]
    uw2 = uw2_ref[...]
    x2 = (_dot(_relu(_dot(x1, uw1) + ub1), uw2) +
          _dot(_relu(_dot(aggr, uw1) + ub1), uw2) + 2.0 * ub2_ref[...])
    x2_ref[...] = x2
    hx1_ref[...] = _relu(_dot(x2, mw1n_ref[...]) + mb1n_ref[...])
    m_ref[...] = jnp.maximum(x1, x2)


def _final_body(x2_ref, m_ref, ga_ref, gb_ref, cnt_ref,
                mw2_ref, mb2_ref, uw1_ref, ub1_ref, uw2_ref, ub2_ref,
                fw_ref, fb_ref, out_ref):
    cnt = cnt_ref[...]
    inv = 1.0 / jnp.maximum(cnt, 1.0)
    ind = (cnt > 0.0).astype(jnp.float32)
    g = (ga_ref[...] + gb_ref[...]) * inv
    aggr = _dot(g, mw2_ref[...]) + 2.0 * ind * mb2_ref[...]
    x2 = x2_ref[...]
    uw1 = uw1_ref[...]
    ub1 = ub1_ref[...]
    uw2 = uw2_ref[...]
    x3 = (_dot(_relu(_dot(x2, uw1) + ub1), uw2) +
          _dot(_relu(_dot(aggr, uw1) + ub1), uw2) + 2.0 * ub2_ref[...])
    m = jnp.maximum(m_ref[...], x3)
    out_ref[...] = _dot(m, fw_ref[...]) + fb_ref[...]


def _row_spec(bn):
    return pl.BlockSpec((bn, F), lambda i: (i, 0))


def _full_spec(shape):
    return pl.BlockSpec(shape, lambda i: (0,) * len(shape))


_W = _full_spec((F, F))
_B = _full_spec((1, F))


def _make_sc_layer(with_cnt):
    out_type = [jax.ShapeDtypeStruct((NC, NP, F), jnp.float32)]
    if with_cnt:
        out_type += [jax.ShapeDtypeStruct((NP,), jnp.float32),
                     jax.ShapeDtypeStruct((NP,), jnp.float32)]

    scratch = [
        pltpu.VMEM((CHUNK,), jnp.int32),
        pltpu.VMEM((CHUNK,), jnp.int32),
        pltpu.VMEM((CHUNK, F), jnp.float32),
        pltpu.VMEM((CHUNK, F), jnp.float32),
        pltpu.VMEM((CHUNK,), jnp.float32),
        pltpu.VMEM_SHARED((NP, F), jnp.float32),
        pltpu.VMEM_SHARED((NP,), jnp.float32),
        pltpu.SemaphoreType.DMA,
    ]

    mesh = plsc.VectorSubcoreMesh(core_axis_name="c", subcore_axis_name="s")

    @functools.partial(pl.kernel, mesh=mesh, out_type=out_type,
                       scratch_types=scratch)
    def sc_layer(src_hbm, dst_hbm, hx_hbm, he_hbm, *refs):
        if with_cnt:
            g_out, cnt0_out, cnt1_out = refs[0], refs[1], refs[2]
            refs = refs[3:]
        else:
            g_out = refs[0]
            refs = refs[1:]
        srcv, dstv, rows, hev, onesv, g_sh, cnt_sh, sem = refs

        cid = lax.axis_index("c")
        sid = lax.axis_index("s")
        wid = sid * NC + cid
        base = wid * EPW

        def zero_rows(i, _):
            def zero_lane(j, _):
                rows[i, pl.ds(j * 16, 16)] = jnp.zeros((16,), jnp.float32)
                return 0
            return lax.fori_loop(0, F // 16, zero_lane, 0)
        lax.fori_loop(0, CHUNK, zero_rows, 0)
        for k in range(STRIPE // CHUNK):
            pltpu.sync_copy(rows, g_sh.at[pl.ds(sid * STRIPE + k * CHUNK, CHUNK)])

        def zero_ones(i, _):
            onesv[pl.ds(i * 16, 16)] = jnp.zeros((16,), jnp.float32)
            return 0
        lax.fori_loop(0, CHUNK // 16, zero_ones, 0)
        for k in range(STRIPE // CHUNK):
            pltpu.sync_copy(onesv, cnt_sh.at[pl.ds(sid * STRIPE + k * CHUNK, CHUNK)])

        def set_ones(i, _):
            onesv[pl.ds(i * 16, 16)] = jnp.ones((16,), jnp.float32)
            return 0
        lax.fori_loop(0, CHUNK // 16, set_ones, 0)

        plsc.subcore_barrier()

        def chunk_body(cix, _):
            eb = base + cix * CHUNK
            pltpu.sync_copy(src_hbm.at[pl.ds(eb, CHUNK)], srcv)
            gather = pltpu.async_copy(hx_hbm.at[srcv], rows, sem)
            pltpu.sync_copy(dst_hbm.at[pl.ds(eb, CHUNK)], dstv)
            pltpu.sync_copy(he_hbm.at[pl.ds(eb, CHUNK)], hev)
            gather.wait()
            pltpu.sync_copy(rows, g_sh.at[dstv], add=True)
            pltpu.sync_copy(hev, g_sh.at[dstv], add=True)
            if with_cnt:
                pltpu.sync_copy(onesv, cnt_sh.at[dstv], add=True)
            return 0
        lax.fori_loop(0, NCHUNKS, chunk_body, 0)

        plsc.subcore_barrier()

        pltpu.sync_copy(g_sh.at[pl.ds(sid * STRIPE, STRIPE)],
                        g_out.at[cid, pl.ds(sid * STRIPE, STRIPE)])
        if with_cnt:
            @pl.when(cid == 0)
            def _():
                pltpu.sync_copy(cnt_sh.at[pl.ds(sid * STRIPE, STRIPE)],
                                cnt0_out.at[pl.ds(sid * STRIPE, STRIPE)])

            @pl.when(cid == 1)
            def _():
                pltpu.sync_copy(cnt_sh.at[pl.ds(sid * STRIPE, STRIPE)],
                                cnt1_out.at[pl.ds(sid * STRIPE, STRIPE)])

    return sc_layer


_sc_layer_cnt = _make_sc_layer(True)
_sc_layer_nocnt = _make_sc_layer(False)


def kernel(x, edge_index, edge_attr, node_w, node_b, edge_w, edge_b,
           mp0_msg_w1, mp0_msg_b1, mp0_msg_w2, mp0_msg_b2,
           mp0_upd_w1, mp0_upd_b1, mp0_upd_w2, mp0_upd_b2,
           mp1_msg_w1, mp1_msg_b1, mp1_msg_w2, mp1_msg_b2,
           mp1_upd_w1, mp1_upd_b1, mp1_upd_w2, mp1_upd_b2,
           final_w, final_b):
    x_p = jnp.pad(x, ((0, NP - N), (0, 0)))
    src = edge_index[0]
    dst = edge_index[1]
    r2 = lambda b: b.reshape(1, F)

    x1, hx0 = pl.pallas_call(
        _node_prep_body,
        grid=(GRID_N,),
        in_specs=[_row_spec(BN), _W, _B, _W, _B],
        out_specs=[_row_spec(BN), _row_spec(BN)],
        out_shape=[jax.ShapeDtypeStruct((NP, F), jnp.float32),
                   jax.ShapeDtypeStruct((NP, F), jnp.float32)],
    )(x_p, node_w, r2(node_b), mp0_msg_w1, r2(mp0_msg_b1))

    he0, he1 = pl.pallas_call(
        _edge_prep_body,
        grid=(GRID_E,),
        in_specs=[pl.BlockSpec((BE, DE), lambda i: (i, 0)),
                  _full_spec((DE, F)), _B, _W, _B, _W, _B],
        out_specs=[_row_spec(BE), _row_spec(BE)],
        out_shape=[jax.ShapeDtypeStruct((E, F), jnp.float32),
                   jax.ShapeDtypeStruct((E, F), jnp.float32)],
    )(edge_attr, edge_w, r2(edge_b), mp0_msg_w1, r2(mp0_msg_b1),
      mp1_msg_w1, r2(mp1_msg_b1))

    g0, cnt0a, cnt0b = _sc_layer_cnt(src, dst, hx0, he0)
    cnt_col = (cnt0a + cnt0b)[:, None]

    cnt_spec = pl.BlockSpec((BN, 1), lambda i: (i, 0))
    x2, hx1, m = pl.pallas_call(
        _mid_body,
        grid=(GRID_N,),
        in_specs=[_row_spec(BN), _row_spec(BN), _row_spec(BN), cnt_spec,
                  _W, _B, _W, _B, _W, _B, _W, _B],
        out_specs=[_row_spec(BN), _row_spec(BN), _row_spec(BN)],
        out_shape=[jax.ShapeDtypeStruct((NP, F), jnp.float32),
                   jax.ShapeDtypeStruct((NP, F), jnp.float32),
                   jax.ShapeDtypeStruct((NP, F), jnp.float32)],
    )(x1, g0[0], g0[1], cnt_col,
      mp0_msg_w2, r2(mp0_msg_b2), mp0_upd_w1, r2(mp0_upd_b1),
      mp0_upd_w2, r2(mp0_upd_b2), mp1_msg_w1, r2(mp1_msg_b1))

    g1 = _sc_layer_nocnt(src, dst, hx1, he1)
    if isinstance(g1, (list, tuple)):
        g1 = g1[0]

    out = pl.pallas_call(
        _final_body,
        grid=(GRID_N,),
        in_specs=[_row_spec(BN), _row_spec(BN), _row_spec(BN), _row_spec(BN),
                  cnt_spec, _W, _B, _W, _B, _W, _B, _W, _B],
        out_specs=_row_spec(BN),
        out_shape=jax.ShapeDtypeStruct((NP, F), jnp.float32),
    )(x2, m, g1[0], g1[1], cnt_col,
      mp1_msg_w2, r2(mp1_msg_b2), mp1_upd_w1, r2(mp1_upd_b1),
      mp1_upd_w2, r2(mp1_upd_b2), final_w, r2(final_b))

    return out[:N]


# speedup vs baseline: 5.1733x; 1.4578x over previous
import functools

import jax
import jax.numpy as jnp
from jax import lax
from jax.experimental import pallas as pl
from jax.experimental.pallas import tpu as pltpu
from jax.experimental.pallas import tpu_sc as plsc

N = 10000
NP = 10240
E = 320000
DIN = 128
DE = 16
F = 128

NC = 2
NS = 16
NW = NC * NS
EPW = E // NW
CHUNK = 80
NCHUNKS = EPW // CHUNK
STRIPE = NP // NS

BN = 512
GRID_N = NP // BN
BE = 4000
GRID_E = E // BE


def _relu(t):
    return jnp.maximum(t, 0.0)


def _dot(a, b):
    return jnp.dot(a, b, preferred_element_type=jnp.float32)


def _node_prep_body(x_ref, nw_ref, nb_ref, mw1_ref, mb1_ref, x1_ref, hx0_ref):
    x1 = _dot(x_ref[...], nw_ref[...]) + nb_ref[...]
    x1_ref[...] = x1
    hx0_ref[...] = _relu(_dot(x1, mw1_ref[...]) + mb1_ref[...])


def _edge_prep_body(ea_ref, ew_ref, eb_ref, mw1_ref, mb1_ref, he_ref):
    ea = ea_ref[...]
    ew = ew_ref[...]
    eb = eb_ref[...]
    a = _dot(ew, mw1_ref[...])
    c = _dot(eb, mw1_ref[...]) + mb1_ref[...]
    he_ref[...] = _relu(_dot(ea, a) + c)


def _mid_body(x1_ref, ga_ref, gb_ref, cnt_ref,
              mw2_ref, mb2_ref, uw1_ref, ub1_ref, uw2_ref, ub2_ref,
              mw1n_ref, mb1n_ref,
              x2_ref, hx1_ref, m_ref):
    cnt = cnt_ref[...]
    inv = 1.0 / jnp.maximum(cnt, 1.0)
    ind = (cnt > 0.0).astype(jnp.float32)
    g = (ga_ref[...] + gb_ref[...]) * inv
    aggr = _dot(g, mw2_ref[...]) + 2.0 * ind * mb2_ref[...]
    x1 = x1_ref[...]
    uw1 = uw1_ref[...]
    ub1 = ub1_ref[...]
    uw2 = uw2_ref[...]
    x2 = (_dot(_relu(_dot(x1, uw1) + ub1), uw2) +
          _dot(_relu(_dot(aggr, uw1) + ub1), uw2) + 2.0 * ub2_ref[...])
    x2_ref[...] = x2
    hx1_ref[...] = _relu(_dot(x2, mw1n_ref[...]) + mb1n_ref[...])
    m_ref[...] = jnp.maximum(x1, x2)


def _final_body(x2_ref, m_ref, ga_ref, gb_ref, cnt_ref,
                mw2_ref, mb2_ref, uw1_ref, ub1_ref, uw2_ref, ub2_ref,
                fw_ref, fb_ref, out_ref):
    cnt = cnt_ref[...]
    inv = 1.0 / jnp.maximum(cnt, 1.0)
    ind = (cnt > 0.0).astype(jnp.float32)
    g = (ga_ref[...] + gb_ref[...]) * inv
    aggr = _dot(g, mw2_ref[...]) + 2.0 * ind * mb2_ref[...]
    x2 = x2_ref[...]
    uw1 = uw1_ref[...]
    ub1 = ub1_ref[...]
    uw2 = uw2_ref[...]
    x3 = (_dot(_relu(_dot(x2, uw1) + ub1), uw2) +
          _dot(_relu(_dot(aggr, uw1) + ub1), uw2) + 2.0 * ub2_ref[...])
    m = jnp.maximum(m_ref[...], x3)
    out_ref[...] = _dot(m, fw_ref[...]) + fb_ref[...]


def _row_spec(bn):
    return pl.BlockSpec((bn, F), lambda i: (i, 0))


def _full_spec(shape):
    return pl.BlockSpec(shape, lambda i: (0,) * len(shape))


_W = _full_spec((F, F))
_B = _full_spec((1, F))


def _make_sc_layer(with_cnt):
    out_type = [jax.ShapeDtypeStruct((NC, NP, F), jnp.float32)]
    if with_cnt:
        out_type += [jax.ShapeDtypeStruct((NP,), jnp.float32),
                     jax.ShapeDtypeStruct((NP,), jnp.float32)]

    scratch = (
        [pltpu.VMEM((CHUNK,), jnp.int32) for _ in range(4)] +
        [pltpu.VMEM((CHUNK,), jnp.int32) for _ in range(4)] +
        [pltpu.VMEM((CHUNK, F), jnp.float32) for _ in range(2)] +
        [pltpu.VMEM((CHUNK, F), jnp.float32) for _ in range(2)] +
        [pltpu.VMEM((CHUNK,), jnp.float32),
         pltpu.VMEM_SHARED((NP, F), jnp.float32),
         pltpu.VMEM_SHARED((NP,), jnp.float32)] +
        [pltpu.SemaphoreType.DMA for _ in range(8)]
    )

    mesh = plsc.VectorSubcoreMesh(core_axis_name="c", subcore_axis_name="s")

    @functools.partial(pl.kernel, mesh=mesh, out_type=out_type,
                       scratch_types=scratch)
    def sc_layer(src_hbm, dst_hbm, hx_hbm, he_hbm, *refs):
        if with_cnt:
            g_out, cnt0_out, cnt1_out = refs[0], refs[1], refs[2]
            refs = refs[3:]
        else:
            g_out = refs[0]
            refs = refs[1:]
        srcv = refs[0:4]
        dstv = refs[4:8]
        rowb = refs[8:10]
        heb = refs[10:12]
        onesv, g_sh, cnt_sh = refs[12:15]
        semidx = refs[15:19]
        semdat = refs[19:21]
        semsc = refs[21:23]
        rows0 = rowb[0]

        cid = lax.axis_index("c")
        sid = lax.axis_index("s")
        wid = sid * NC + cid

        def zero_rows(i, _):
            def zero_lane(j, _):
                rows0[i, pl.ds(j * 16, 16)] = jnp.zeros((16,), jnp.float32)
                return 0
            return lax.fori_loop(0, F // 16, zero_lane, 0)
        lax.fori_loop(0, CHUNK, zero_rows, 0)
        for k in range(STRIPE // CHUNK):
            pltpu.sync_copy(rows0, g_sh.at[pl.ds(sid * STRIPE + k * CHUNK, CHUNK)])

        def zero_ones(i, _):
            onesv[pl.ds(i * 16, 16)] = jnp.zeros((16,), jnp.float32)
            return 0
        lax.fori_loop(0, CHUNK // 16, zero_ones, 0)
        for k in range(STRIPE // CHUNK):
            pltpu.sync_copy(onesv, cnt_sh.at[pl.ds(sid * STRIPE + k * CHUNK, CHUNK)])

        def set_ones(i, _):
            onesv[pl.ds(i * 16, 16)] = jnp.ones((16,), jnp.float32)
            return 0
        lax.fori_loop(0, CHUNK // 16, set_ones, 0)

        plsc.subcore_barrier()

        ebase = wid * EPW

        def idx_load(cix, s4):
            b = ebase + cix * CHUNK
            pltpu.async_copy(src_hbm.at[pl.ds(b, CHUNK)], srcv[s4], semidx[s4])
            pltpu.async_copy(dst_hbm.at[pl.ds(b, CHUNK)], dstv[s4], semidx[s4])

        def wait_idx(s4):
            pltpu.make_async_copy(src_hbm.at[pl.ds(0, CHUNK)], srcv[s4],
                                  semidx[s4]).wait()
            pltpu.make_async_copy(dst_hbm.at[pl.ds(0, CHUNK)], dstv[s4],
                                  semidx[s4]).wait()

        def data_load(cix, s4, s2):
            pltpu.async_copy(hx_hbm.at[srcv[s4]], rowb[s2], semdat[s2])
            pltpu.async_copy(he_hbm.at[pl.ds(ebase + cix * CHUNK, CHUNK)],
                             heb[s2], semdat[s2])

        def wait_data(s2):
            pltpu.make_async_copy(hx_hbm.at[srcv[0]], rowb[s2],
                                  semdat[s2]).wait()
            pltpu.make_async_copy(he_hbm.at[pl.ds(0, CHUNK)], heb[s2],
                                  semdat[s2]).wait()

        def scatter(s4, s2):
            pltpu.async_copy(rowb[s2], g_sh.at[dstv[s4]], semsc[s2], add=True)
            pltpu.async_copy(heb[s2], g_sh.at[dstv[s4]], semsc[s2], add=True)
            if with_cnt:
                pltpu.async_copy(onesv, cnt_sh.at[dstv[s4]], semsc[s2],
                                 add=True)

        def wait_scatter(s2):
            pltpu.make_async_copy(rowb[s2], g_sh.at[dstv[0]], semsc[s2]).wait()
            pltpu.make_async_copy(heb[s2], g_sh.at[dstv[0]], semsc[s2]).wait()
            if with_cnt:
                pltpu.make_async_copy(onesv, cnt_sh.at[dstv[0]],
                                      semsc[s2]).wait()

        idx_load(0, 0)
        idx_load(1, 1)
        wait_idx(0)
        data_load(0, 0, 0)

        def body(i, _):
            for s in range(4):
                @pl.when(i % 4 == s)
                def _(s=s):
                    @pl.when(i >= 1)
                    def _():
                        wait_scatter((s + 1) % 2)

                    @pl.when(i + 2 < NCHUNKS)
                    def _():
                        idx_load(i + 2, (s + 2) % 4)

                    @pl.when(i + 1 < NCHUNKS)
                    def _():
                        wait_idx((s + 1) % 4)
                        data_load(i + 1, (s + 1) % 4, (s + 1) % 2)

                    wait_data(s % 2)
                    scatter(s % 4, s % 2)
            return 0
        lax.fori_loop(0, NCHUNKS, body, 0)

        wait_scatter((NCHUNKS - 1) % 2)

        plsc.subcore_barrier()

        pltpu.sync_copy(g_sh.at[pl.ds(sid * STRIPE, STRIPE)],
                        g_out.at[cid, pl.ds(sid * STRIPE, STRIPE)])
        if with_cnt:
            @pl.when(cid == 0)
            def _():
                pltpu.sync_copy(cnt_sh.at[pl.ds(sid * STRIPE, STRIPE)],
                                cnt0_out.at[pl.ds(sid * STRIPE, STRIPE)])

            @pl.when(cid == 1)
            def _():
                pltpu.sync_copy(cnt_sh.at[pl.ds(sid * STRIPE, STRIPE)],
                                cnt1_out.at[pl.ds(sid * STRIPE, STRIPE)])

    return sc_layer


_sc_layer_cnt = _make_sc_layer(True)
_sc_layer_nocnt = _make_sc_layer(False)


def kernel(x, edge_index, edge_attr, node_w, node_b, edge_w, edge_b,
           mp0_msg_w1, mp0_msg_b1, mp0_msg_w2, mp0_msg_b2,
           mp0_upd_w1, mp0_upd_b1, mp0_upd_w2, mp0_upd_b2,
           mp1_msg_w1, mp1_msg_b1, mp1_msg_w2, mp1_msg_b2,
           mp1_upd_w1, mp1_upd_b1, mp1_upd_w2, mp1_upd_b2,
           final_w, final_b):
    x_p = jnp.pad(x, ((0, NP - N), (0, 0)))
    src = edge_index[0]
    dst = edge_index[1]
    r2 = lambda b: b.reshape(1, F)

    x1, hx0 = pl.pallas_call(
        _node_prep_body,
        grid=(GRID_N,),
        in_specs=[_row_spec(BN), _W, _B, _W, _B],
        out_specs=[_row_spec(BN), _row_spec(BN)],
        out_shape=[jax.ShapeDtypeStruct((NP, F), jnp.float32),
                   jax.ShapeDtypeStruct((NP, F), jnp.float32)],
    )(x_p, node_w, r2(node_b), mp0_msg_w1, r2(mp0_msg_b1))

    def edge_prep(mw1, mb1):
        return pl.pallas_call(
            _edge_prep_body,
            grid=(GRID_E,),
            in_specs=[pl.BlockSpec((BE, DE), lambda i: (i, 0)),
                      _full_spec((DE, F)), _B, _W, _B],
            out_specs=_row_spec(BE),
            out_shape=jax.ShapeDtypeStruct((E, F), jnp.float32),
        )(edge_attr, edge_w, r2(edge_b), mw1, r2(mb1))

    he0 = edge_prep(mp0_msg_w1, mp0_msg_b1)
    he1 = edge_prep(mp1_msg_w1, mp1_msg_b1)

    g0, cnt0a, cnt0b = _sc_layer_cnt(src, dst, hx0, he0)
    cnt_col = (cnt0a + cnt0b)[:, None]

    cnt_spec = pl.BlockSpec((BN, 1), lambda i: (i, 0))
    x2, hx1, m = pl.pallas_call(
        _mid_body,
        grid=(GRID_N,),
        in_specs=[_row_spec(BN), _row_spec(BN), _row_spec(BN), cnt_spec,
                  _W, _B, _W, _B, _W, _B, _W, _B],
        out_specs=[_row_spec(BN), _row_spec(BN), _row_spec(BN)],
        out_shape=[jax.ShapeDtypeStruct((NP, F), jnp.float32),
                   jax.ShapeDtypeStruct((NP, F), jnp.float32),
                   jax.ShapeDtypeStruct((NP, F), jnp.float32)],
    )(x1, g0[0], g0[1], cnt_col,
      mp0_msg_w2, r2(mp0_msg_b2), mp0_upd_w1, r2(mp0_upd_b1),
      mp0_upd_w2, r2(mp0_upd_b2), mp1_msg_w1, r2(mp1_msg_b1))

    g1 = _sc_layer_nocnt(src, dst, hx1, he1)
    if isinstance(g1, (list, tuple)):
        g1 = g1[0]

    out = pl.pallas_call(
        _final_body,
        grid=(GRID_N,),
        in_specs=[_row_spec(BN), _row_spec(BN), _row_spec(BN), _row_spec(BN),
                  cnt_spec, _W, _B, _W, _B, _W, _B, _W, _B],
        out_specs=_row_spec(BN),
        out_shape=jax.ShapeDtypeStruct((NP, F), jnp.float32),
    )(x2, m, g1[0], g1[1], cnt_col,
      mp1_msg_w2, r2(mp1_msg_b2), mp1_upd_w1, r2(mp1_upd_b1),
      mp1_upd_w2, r2(mp1_upd_b2), final_w, r2(final_b))

    return out[:N]
